# baseline (speedup 1.0000x reference)
import jax
import jax.numpy as jnp
from jax.experimental import pallas as pl
from jax.experimental.pallas import tpu as pltpu

_EPS = 1e-12
_HC = 8


def _embed_ln_kernel(grid_ref, row_ref, col_ref, tt_ref, gamma_ref, beta_ref,
                     out_ref):
    g = grid_ref[0]
    nfrm = g.shape[0]
    x = jnp.sum(g, axis=0) * (1.0 / nfrm)
    x = x + row_ref[...][:, None, :] + col_ref[...][None, :, :]
    x = x + tt_ref[...][None, :, :]
    mu = jnp.mean(x, axis=-1, keepdims=True)
    var = jnp.mean(jnp.square(x - mu), axis=-1, keepdims=True)
    xhat = (x - mu) * jax.lax.rsqrt(var + _EPS)
    y = xhat * gamma_ref[...][None, :, :] + beta_ref[...][None, :, :]
    out_ref[0] = y.reshape(out_ref.shape[1], out_ref.shape[2])


def kernel(grid, row_table, col_table, tt_table, gamma, beta):
    B, NFRM, H, W, D = grid.shape
    gamma2 = gamma.reshape(1, D)
    beta2 = beta.reshape(1, D)
    out = pl.pallas_call(
        _embed_ln_kernel,
        grid=(B, H // _HC),
        in_specs=[
            pl.BlockSpec((1, NFRM, _HC, W, D), lambda b, h: (b, 0, h, 0, 0)),
            pl.BlockSpec((_HC, D), lambda b, h: (h, 0)),
            pl.BlockSpec((W, D), lambda b, h: (0, 0)),
            pl.BlockSpec((1, D), lambda b, h: (0, 0)),
            pl.BlockSpec((1, D), lambda b, h: (0, 0)),
            pl.BlockSpec((1, D), lambda b, h: (0, 0)),
        ],
        out_specs=pl.BlockSpec((1, _HC * W, D), lambda b, h: (b, h, 0)),
        out_shape=jax.ShapeDtypeStruct((B, H * W, D), grid.dtype),
        compiler_params=pltpu.CompilerParams(
            dimension_semantics=("parallel", "parallel"),
        ),
    )(grid, row_table, col_table, tt_table, gamma2, beta2)
    return out


# speedup vs baseline: 1.0343x; 1.0343x over previous
import jax
import jax.numpy as jnp
from jax.experimental import pallas as pl
from jax.experimental.pallas import tpu as pltpu

_EPS = 1e-12
_HC = 24


def _embed_ln_kernel(grid_ref, row_ref, col_ref, tt_ref, gamma_ref, beta_ref,
                     out_ref):
    g = grid_ref[0]
    nfrm = g.shape[0]
    x = jnp.sum(g, axis=0) * (1.0 / nfrm)
    x = x + row_ref[...][:, None, :] + col_ref[...][None, :, :]
    x = x + tt_ref[...][None, :, :]
    mu = jnp.mean(x, axis=-1, keepdims=True)
    var = jnp.mean(jnp.square(x - mu), axis=-1, keepdims=True)
    xhat = (x - mu) * jax.lax.rsqrt(var + _EPS)
    y = xhat * gamma_ref[...][None, :, :] + beta_ref[...][None, :, :]
    out_ref[0] = y.reshape(out_ref.shape[1], out_ref.shape[2])


def kernel(grid, row_table, col_table, tt_table, gamma, beta):
    B, NFRM, H, W, D = grid.shape
    gamma2 = gamma.reshape(1, D)
    beta2 = beta.reshape(1, D)
    out = pl.pallas_call(
        _embed_ln_kernel,
        grid=(B, H // _HC),
        in_specs=[
            pl.BlockSpec((1, NFRM, _HC, W, D), lambda b, h: (b, 0, h, 0, 0)),
            pl.BlockSpec((_HC, D), lambda b, h: (h, 0)),
            pl.BlockSpec((W, D), lambda b, h: (0, 0)),
            pl.BlockSpec((1, D), lambda b, h: (0, 0)),
            pl.BlockSpec((1, D), lambda b, h: (0, 0)),
            pl.BlockSpec((1, D), lambda b, h: (0, 0)),
        ],
        out_specs=pl.BlockSpec((1, _HC * W, D), lambda b, h: (b, h, 0)),
        out_shape=jax.ShapeDtypeStruct((B, H * W, D), grid.dtype),
        compiler_params=pltpu.CompilerParams(
            dimension_semantics=("parallel", "parallel"),
        ),
    )(grid, row_table, col_table, tt_table, gamma2, beta2)
    return out
